# baseline (speedup 1.0000x reference)
import jax
import jax.numpy as jnp
from jax.experimental import pallas as pl

B, C_IN, N = 16, 128, 1024
C_OUT = 256
BPS = 4


def _gnn_body(x_ref, a_ref, w_ref, b_ref, y_ref):
    a0 = a_ref[0].astype(jnp.bfloat16)
    a1 = a_ref[1].astype(jnp.bfloat16)
    xb = x_ref[...].reshape(BPS * C_IN, N).astype(jnp.bfloat16)

    h1 = jnp.dot(xb, a0, preferred_element_type=jnp.float32).astype(jnp.bfloat16)
    h3 = jnp.dot(xb, a1, preferred_element_type=jnp.float32).astype(jnp.bfloat16)
    h2 = jnp.dot(h1, a0, preferred_element_type=jnp.float32).astype(jnp.bfloat16)
    h4 = jnp.dot(h3, a1, preferred_element_type=jnp.float32).astype(jnp.bfloat16)

    w16 = w_ref[...].astype(jnp.bfloat16)
    bias = b_ref[...]
    parts = (xb, h1, h2, h3, h4)
    for i in range(BPS):
        s = slice(i * C_IN, (i + 1) * C_IN)
        acc = bias
        for k, p in enumerate(parts):
            wk = w16[:, k * C_IN:(k + 1) * C_IN]
            acc = acc + jnp.dot(wk, p[s], preferred_element_type=jnp.float32)
        y_ref[i] = acc


def kernel(x, adjs, W, b):
    b2d = b.reshape(C_OUT, 1)
    grid = (B // BPS,)
    return pl.pallas_call(
        _gnn_body,
        grid=grid,
        in_specs=[
            pl.BlockSpec((BPS, C_IN, N), lambda i: (i, 0, 0)),
            pl.BlockSpec((2, N, N), lambda i: (0, 0, 0)),
            pl.BlockSpec((C_OUT, 5 * C_IN), lambda i: (0, 0)),
            pl.BlockSpec((C_OUT, 1), lambda i: (0, 0)),
        ],
        out_specs=pl.BlockSpec((BPS, C_OUT, N), lambda i: (i, 0, 0)),
        out_shape=jax.ShapeDtypeStruct((B, C_OUT, N), jnp.float32),
    )(x, adjs, W, b2d)


# speedup vs baseline: 1.0982x; 1.0982x over previous
import jax
import jax.numpy as jnp
from jax.experimental import pallas as pl
from jax.experimental.pallas import tpu as pltpu

B, C_IN, N = 16, 128, 1024
C_OUT = 256
C_CAT = 5 * C_IN
BPS = 4


def _gnn_body(x_ref, a_ref, w_ref, b_ref, y_ref, a16_ref, xc_ref):
    @pl.when(pl.program_id(0) == 0)
    def _cache_adj():
        a16_ref[...] = a_ref[...].astype(jnp.bfloat16)

    a0 = a16_ref[0]
    a1 = a16_ref[1]
    xb = x_ref[...].reshape(BPS * C_IN, N).astype(jnp.bfloat16)

    h1 = jnp.dot(xb, a0, preferred_element_type=jnp.float32).astype(jnp.bfloat16)
    h3 = jnp.dot(xb, a1, preferred_element_type=jnp.float32).astype(jnp.bfloat16)
    h2 = jnp.dot(h1, a0, preferred_element_type=jnp.float32).astype(jnp.bfloat16)
    h4 = jnp.dot(h3, a1, preferred_element_type=jnp.float32).astype(jnp.bfloat16)

    for i in range(BPS):
        s = slice(i * C_IN, (i + 1) * C_IN)
        base = i * C_CAT
        for k, part in enumerate((xb, h1, h2, h3, h4)):
            xc_ref[base + k * C_IN:base + (k + 1) * C_IN, :] = part[s]

    w16 = w_ref[...].astype(jnp.bfloat16)
    bias = b_ref[...]
    for i in range(BPS):
        xc = xc_ref[i * C_CAT:(i + 1) * C_CAT, :]
        y_ref[i] = jnp.dot(w16, xc, preferred_element_type=jnp.float32) + bias


def kernel(x, adjs, W, b):
    b2d = b.reshape(C_OUT, 1)
    grid = (B // BPS,)
    return pl.pallas_call(
        _gnn_body,
        grid=grid,
        in_specs=[
            pl.BlockSpec((BPS, C_IN, N), lambda i: (i, 0, 0)),
            pl.BlockSpec((2, N, N), lambda i: (0, 0, 0)),
            pl.BlockSpec((C_OUT, C_CAT), lambda i: (0, 0)),
            pl.BlockSpec((C_OUT, 1), lambda i: (0, 0)),
        ],
        out_specs=pl.BlockSpec((BPS, C_OUT, N), lambda i: (i, 0, 0)),
        out_shape=jax.ShapeDtypeStruct((B, C_OUT, N), jnp.float32),
        scratch_shapes=[
            pltpu.VMEM((2, N, N), jnp.bfloat16),
            pltpu.VMEM((BPS * C_CAT, N), jnp.bfloat16),
        ],
    )(x, adjs, W, b2d)
